# Initial kernel scaffold; baseline (speedup 1.0000x reference)
import jax
import jax.numpy as jnp
from jax.experimental import pallas as pl


def kernel(node_feats, edge_index, edge_attr, W1, b1, W2, b2, W_emb, b_emb):
    raise NotImplementedError("write your pallas kernel here")


# speedup vs baseline: 4.7832x; 4.7832x over previous
import functools

import jax
import jax.numpy as jnp
from jax import lax
from jax.experimental import pallas as pl
from jax.experimental.pallas import tpu as pltpu
from jax.experimental.pallas import tpu_sc as plsc

N_NODES = 5000
N_EDGES = 320000
IN_NF = 128
EDGE_NF = 16
H_NF = 256
OUT_NF = 128
EMB_NF = 4

_NC = 2
_NS = 16
_NW = _NC * _NS
_EW = N_EDGES // _NW
_CHUNK = 100
_NCH = _EW // _CHUNK
_BLK_CH = 20
_NBLK = _NCH // _BLK_CH
_BLK_E = _BLK_CH * _CHUNK
_N_PAD = 5120
_ROWS_W = _N_PAD // _NS


def _sc_segment_sum(row2d, edge_attr):
    mesh = plsc.VectorSubcoreMesh(core_axis_name="c", subcore_axis_name="s")

    @functools.partial(
        pl.kernel,
        mesh=mesh,
        out_type=jax.ShapeDtypeStruct((_NC, _N_PAD, EDGE_NF), jnp.float32),
        compiler_params=pltpu.CompilerParams(use_tc_tiling_on_sc=False),
        scratch_types=[
            pltpu.VMEM((_NCH, _CHUNK), jnp.int32),
            pltpu.VMEM((_BLK_E, EDGE_NF), jnp.float32),
            pltpu.VMEM((_ROWS_W, EDGE_NF), jnp.float32),
            pltpu.VMEM_SHARED((_N_PAD, EDGE_NF), jnp.float32),
        ],
    )
    def k(idx_hbm, attr_hbm, out_hbm, idx_v, attr_v, zbuf_v, acc_sh):
        c = lax.axis_index("c")
        s = lax.axis_index("s")
        wid = s * _NC + c

        def zero_body(i, _):
            zbuf_v[i, :] = jnp.zeros((EDGE_NF,), jnp.float32)
            return _
        lax.fori_loop(0, _ROWS_W, zero_body, 0)
        pltpu.sync_copy(zbuf_v, acc_sh.at[pl.ds(s * _ROWS_W, _ROWS_W)])
        plsc.subcore_barrier()

        pltpu.sync_copy(idx_hbm.at[wid], idx_v)

        def blk_body(b, _):
            pltpu.sync_copy(
                attr_hbm.at[pl.ds(wid * _EW + b * _BLK_E, _BLK_E)], attr_v)
            for j in range(_BLK_CH):
                pltpu.sync_copy(
                    attr_v.at[pl.ds(j * _CHUNK, _CHUNK)],
                    acc_sh.at[idx_v.at[b * _BLK_CH + j]],
                    add=True,
                )
            return _
        lax.fori_loop(0, _NBLK, blk_body, 0)
        plsc.subcore_barrier()

        pltpu.sync_copy(acc_sh.at[pl.ds(s * _ROWS_W, _ROWS_W)], zbuf_v)
        pltpu.sync_copy(zbuf_v, out_hbm.at[c, pl.ds(s * _ROWS_W, _ROWS_W)])

    return k(row2d, edge_attr)


def _mlp_body(nf_ref, p_ref, w1_ref, b1_ref, w2_ref, b2_ref, we_ref, be_ref,
              emb_ref):
    agg = p_ref[0] + p_ref[1]
    x = jnp.concatenate([nf_ref[...], agg], axis=1)
    h = jnp.maximum(jnp.dot(x, w1_ref[...],
                            preferred_element_type=jnp.float32) + b1_ref[...], 0.0)
    out = jnp.dot(h, w2_ref[...], preferred_element_type=jnp.float32) + b2_ref[...]
    emb_ref[...] = jnp.dot(out, we_ref[...],
                           preferred_element_type=jnp.float32) + be_ref[...]


_DEC_T = 1024


def _dec_body(xr_ref, xc_ref, out_ref):
    i = pl.program_id(0)
    j = pl.program_id(1)
    xr = xr_ref[...]
    xc = xc_ref[...]
    rn = jnp.sum(xr * xr, axis=1, keepdims=True)
    cn = jnp.sum(xc * xc, axis=1, keepdims=True).reshape(1, _DEC_T)
    g = lax.dot_general(xr, xc, (((1,), (1,)), ((), ())),
                        preferred_element_type=jnp.float32)
    d2 = rn + cn - 2.0 * g
    a = jax.nn.sigmoid(3.0 * d2 - 1.0)
    rid = i * _DEC_T + lax.broadcasted_iota(jnp.int32, (_DEC_T, _DEC_T), 0)
    cid = j * _DEC_T + lax.broadcasted_iota(jnp.int32, (_DEC_T, _DEC_T), 1)
    out_ref[...] = jnp.where(rid == cid, 0.0, a)


def kernel(node_feats, edge_index, edge_attr, W1, b1, W2, b2, W_emb, b_emb):
    row2d = edge_index[0].reshape(_NW, _NCH, _CHUNK)
    partials = _sc_segment_sum(row2d, edge_attr)[:, :N_NODES, :]

    node_emb = pl.pallas_call(
        _mlp_body,
        out_shape=jax.ShapeDtypeStruct((N_NODES, EMB_NF), jnp.float32),
    )(node_feats, partials, W1, b1.reshape(1, H_NF), W2,
      b2.reshape(1, OUT_NF), W_emb, b_emb.reshape(1, EMB_NF))

    nt = pl.cdiv(N_NODES, _DEC_T)
    adj = pl.pallas_call(
        _dec_body,
        grid=(nt, nt),
        in_specs=[
            pl.BlockSpec((_DEC_T, EMB_NF), lambda i, j: (i, 0)),
            pl.BlockSpec((_DEC_T, EMB_NF), lambda i, j: (j, 0)),
        ],
        out_specs=pl.BlockSpec((_DEC_T, _DEC_T), lambda i, j: (i, j)),
        out_shape=jax.ShapeDtypeStruct((N_NODES, N_NODES), jnp.float32),
    )(node_emb, node_emb)

    return (node_emb, adj)


# speedup vs baseline: 4.8420x; 1.0123x over previous
import functools

import jax
import jax.numpy as jnp
from jax import lax
from jax.experimental import pallas as pl
from jax.experimental.pallas import tpu as pltpu
from jax.experimental.pallas import tpu_sc as plsc

N_NODES = 5000
N_EDGES = 320000
IN_NF = 128
EDGE_NF = 16
H_NF = 256
OUT_NF = 128
EMB_NF = 4

_NC = 2
_NS = 16
_NW = _NC * _NS
_EW = N_EDGES // _NW
_CHUNK = 100
_NCH = _EW // _CHUNK
_BLK_CH = 20
_NBLK = _NCH // _BLK_CH
_BLK_E = _BLK_CH * _CHUNK
_N_PAD = 5120
_ROWS_W = _N_PAD // _NS


def _sc_segment_sum(edge_index3, edge_attr):
    mesh = plsc.VectorSubcoreMesh(core_axis_name="c", subcore_axis_name="s")

    @functools.partial(
        pl.kernel,
        mesh=mesh,
        out_type=jax.ShapeDtypeStruct((_NC, _N_PAD, EDGE_NF), jnp.float32),
        compiler_params=pltpu.CompilerParams(use_tc_tiling_on_sc=False),
        scratch_types=[
            pltpu.VMEM((_NCH, _CHUNK), jnp.int32),
            pltpu.VMEM((_BLK_E, EDGE_NF), jnp.float32),
            pltpu.VMEM((_ROWS_W, EDGE_NF), jnp.float32),
            pltpu.VMEM_SHARED((_N_PAD, EDGE_NF), jnp.float32),
        ],
    )
    def k(idx_hbm, attr_hbm, out_hbm, idx_v, attr_v, zbuf_v, acc_sh):
        c = lax.axis_index("c")
        s = lax.axis_index("s")
        wid = s * _NC + c

        def zero_body(i, _):
            zbuf_v[i, :] = jnp.zeros((EDGE_NF,), jnp.float32)
            return _
        lax.fori_loop(0, _ROWS_W, zero_body, 0)
        pltpu.sync_copy(zbuf_v, acc_sh.at[pl.ds(s * _ROWS_W, _ROWS_W)])
        plsc.subcore_barrier()

        pltpu.sync_copy(idx_hbm.at[0, pl.ds(wid * _NCH, _NCH)], idx_v)

        def blk_body(b, _):
            pltpu.sync_copy(
                attr_hbm.at[pl.ds(wid * _EW + b * _BLK_E, _BLK_E)], attr_v)
            for j in range(_BLK_CH):
                pltpu.sync_copy(
                    attr_v.at[pl.ds(j * _CHUNK, _CHUNK)],
                    acc_sh.at[idx_v.at[b * _BLK_CH + j]],
                    add=True,
                )
            return _
        lax.fori_loop(0, _NBLK, blk_body, 0)
        plsc.subcore_barrier()

        pltpu.sync_copy(acc_sh.at[pl.ds(s * _ROWS_W, _ROWS_W)], zbuf_v)
        pltpu.sync_copy(zbuf_v, out_hbm.at[c, pl.ds(s * _ROWS_W, _ROWS_W)])

    return k(edge_index3, edge_attr)


_DEC_T = 1024
_LOG2E = 1.4426950408889634


def _tc_body(nf_ref, p_ref, w1_ref, b1_ref, w2_ref, b2_ref, we_ref, be_ref,
             emb_ref, adj_ref, emb_s):
    i = pl.program_id(0)
    j = pl.program_id(1)

    @pl.when((i == 0) & (j == 0))
    def _mlp():
        agg = p_ref[0] + p_ref[1]
        x = jnp.concatenate([nf_ref[...], agg], axis=1)
        h = jnp.maximum(
            jnp.dot(x, w1_ref[...], preferred_element_type=jnp.float32)
            + b1_ref[...], 0.0)
        out = (jnp.dot(h, w2_ref[...], preferred_element_type=jnp.float32)
               + b2_ref[...])
        emb = (jnp.dot(out, we_ref[...], preferred_element_type=jnp.float32)
               + be_ref[...])
        emb_s[pl.ds(0, N_NODES), :] = emb
        emb_ref[...] = emb

    xr = emb_s[pl.ds(i * _DEC_T, _DEC_T), :]
    xc = emb_s[pl.ds(j * _DEC_T, _DEC_T), :]
    rn = jnp.sum(xr * xr, axis=1, keepdims=True)
    cn = jnp.sum(xc * xc, axis=1, keepdims=True).reshape(1, _DEC_T)
    g = lax.dot_general(xr, xc, (((1,), (1,)), ((), ())),
                        preferred_element_type=jnp.float32)
    d2 = rn + cn - 2.0 * g
    e = jnp.exp2((1.0 - 3.0 * d2) * _LOG2E)
    a = 1.0 / (1.0 + e)

    @pl.when(i != j)
    def _off_diag():
        adj_ref[...] = a

    @pl.when(i == j)
    def _diag():
        rid = lax.broadcasted_iota(jnp.int32, (_DEC_T, _DEC_T), 0)
        cid = lax.broadcasted_iota(jnp.int32, (_DEC_T, _DEC_T), 1)
        adj_ref[...] = jnp.where(rid == cid, 0.0, a)


def kernel(node_feats, edge_index, edge_attr, W1, b1, W2, b2, W_emb, b_emb):
    edge_index3 = edge_index.reshape(2, _NW * _NCH, _CHUNK)
    partials = _sc_segment_sum(edge_index3, edge_attr)

    nt = pl.cdiv(N_NODES, _DEC_T)
    const = lambda i, j: (0, 0)
    const3 = lambda i, j: (0, 0, 0)
    node_emb, adj = pl.pallas_call(
        _tc_body,
        grid=(nt, nt),
        in_specs=[
            pl.BlockSpec((N_NODES, IN_NF), const),
            pl.BlockSpec((_NC, N_NODES, EDGE_NF), const3),
            pl.BlockSpec((IN_NF + EDGE_NF, H_NF), const),
            pl.BlockSpec((1, H_NF), const),
            pl.BlockSpec((H_NF, OUT_NF), const),
            pl.BlockSpec((1, OUT_NF), const),
            pl.BlockSpec((OUT_NF, EMB_NF), const),
            pl.BlockSpec((1, EMB_NF), const),
        ],
        out_specs=[
            pl.BlockSpec((N_NODES, EMB_NF), const),
            pl.BlockSpec((_DEC_T, _DEC_T), lambda i, j: (i, j)),
        ],
        out_shape=[
            jax.ShapeDtypeStruct((N_NODES, EMB_NF), jnp.float32),
            jax.ShapeDtypeStruct((N_NODES, N_NODES), jnp.float32),
        ],
        scratch_shapes=[pltpu.VMEM((_N_PAD, EMB_NF), jnp.float32)],
    )(node_feats, partials, W1, b1.reshape(1, H_NF), W2,
      b2.reshape(1, OUT_NF), W_emb, b_emb.reshape(1, EMB_NF))

    return (node_emb, adj)
